# baseline (speedup 1.0000x reference)
import functools

import jax
import jax.numpy as jnp
from jax import lax
from jax.experimental import pallas as pl
from jax.experimental.pallas import tpu as pltpu
from jax.experimental.pallas import tpu_sc as plsc

N = 10000
E = 320000
D = 128

NC = 2
NS = 16
NW = NC * NS
EPW = E // NW
CHUNK = 80
NCHUNK = EPW // CHUNK
RPT = 624
TAIL = N - RPT * NS


def _striped_copy(s, copy):
    copy(s * RPT, RPT)

    @pl.when(s == NS - 1)
    def _():
        copy(NS * RPT, TAIL)

_MESH = plsc.VectorSubcoreMesh(core_axis_name="c", subcore_axis_name="s")


_SC_AGG_KW = dict(
    out_type=jax.ShapeDtypeStruct((NC * N, D), jnp.float32),
    mesh=_MESH,
    scratch_types=[
        pltpu.VMEM((EPW,), jnp.int32),
        pltpu.VMEM((NCHUNK, CHUNK), jnp.int32),
        pltpu.VMEM((CHUNK, D), jnp.float32),
        pltpu.VMEM((CHUNK, D), jnp.float32),
        pltpu.VMEM_SHARED((N, D), jnp.float32),
        pltpu.SemaphoreType.DMA,
        pltpu.SemaphoreType.DMA,
    ],
)


def _agg_step(h, srcv, dstv, aggsh, j, cur, csem, nxt, nsem):
    @pl.when(j + 1 < NCHUNK)
    def _():
        pltpu.async_copy(h.at[srcv.at[pl.ds((j + 1) * CHUNK, CHUNK)]], nxt, nsem)

    pltpu.make_async_copy(h.at[srcv.at[pl.ds(j * CHUNK, CHUNK)]], cur, csem).wait()
    pltpu.sync_copy(cur, aggsh.at[dstv.at[j]], add=True)


_DEG_WND = 32


def _agg_impl(h, src, dst, zeros, agg_out, deg_out,
              srcv, dstv, rows0, rows1, aggsh, gsem0, gsem1, ssem):
    c = lax.axis_index("c")
    s = lax.axis_index("s")
    wid = s * NC + c
    pltpu.sync_copy(src.at[wid], srcv)
    pltpu.sync_copy(dst.at[wid], dstv)
    _striped_copy(s, lambda r0, nr: pltpu.sync_copy(
        zeros.at[pl.ds(r0, nr)], aggsh.at[pl.ds(r0, nr)]))

    if deg_out is not None:
        def fill(i, carry):
            for k in range(D // 16):
                rows0[i, pl.ds(k * 16, 16)] = jnp.ones((16,), jnp.float32)
            return carry

        lax.fori_loop(0, CHUNK, fill, 0)
        plsc.subcore_barrier()

        def dstep(j, carry):
            pltpu.async_copy(rows0, aggsh.at[dstv.at[j]], ssem, add=True)

            @pl.when(j >= _DEG_WND)
            def _():
                pltpu.make_async_copy(rows0, aggsh.at[dstv.at[0]], ssem).wait()

            return carry

        lax.fori_loop(0, NCHUNK, dstep, 0)

        def ddrain(j, carry):
            pltpu.make_async_copy(rows0, aggsh.at[dstv.at[0]], ssem).wait()
            return carry

        lax.fori_loop(0, _DEG_WND, ddrain, 0)
        plsc.subcore_barrier()
        _striped_copy(s, lambda r0, nr: pltpu.sync_copy(
            aggsh.at[pl.ds(r0, nr)], deg_out.at[pl.ds(c * N + r0, nr)]))

    pltpu.async_copy(h.at[srcv.at[pl.ds(0, CHUNK)]], rows0, gsem0)
    plsc.subcore_barrier()

    def pair(p, carry):
        j = 2 * p
        _agg_step(h, srcv, dstv, aggsh, j, rows0, gsem0, rows1, gsem1)
        _agg_step(h, srcv, dstv, aggsh, j + 1, rows1, gsem1, rows0, gsem0)
        return carry

    lax.fori_loop(0, NCHUNK // 2, pair, 0)
    if NCHUNK % 2:
        _agg_step(h, srcv, dstv, aggsh, NCHUNK - 1, rows0, gsem0, rows1, gsem1)
    plsc.subcore_barrier()
    _striped_copy(s, lambda r0, nr: pltpu.sync_copy(
        aggsh.at[pl.ds(r0, nr)], agg_out.at[pl.ds(c * N + r0, nr)]))


def _sc_agg_body(h, src, dst, zeros, agg_out,
                 srcv, dstv, rows0, rows1, aggsh, gsem0, gsem1):
    _agg_impl(h, src, dst, zeros, agg_out, None,
              srcv, dstv, rows0, rows1, aggsh, gsem0, gsem1, None)


def _sc_agg_deg_body(h, src, dst, zeros, agg_out, deg_out,
                     srcv, dstv, rows0, rows1, aggsh, gsem0, gsem1, ssem):
    _agg_impl(h, src, dst, zeros, agg_out, deg_out,
              srcv, dstv, rows0, rows1, aggsh, gsem0, gsem1, ssem)


_sc_agg = pl.kernel(_sc_agg_body, **_SC_AGG_KW)

_SC_AGG_DEG_KW = dict(
    out_type=(
        jax.ShapeDtypeStruct((NC * N, D), jnp.float32),
        jax.ShapeDtypeStruct((NC * N, D), jnp.float32),
    ),
    mesh=_MESH,
    scratch_types=list(_SC_AGG_KW["scratch_types"]) + [pltpu.SemaphoreType.DMA],
)

_sc_agg_deg = pl.kernel(_sc_agg_deg_body, **_SC_AGG_DEG_KW)


BLK = 2000
NBLK = N // BLK


def _make_tc_body(sub_deg):
    def _tc_body(x_ref, a0_ref, a1_ref, d0_ref, d1_ref,
                 ws_ref, wn_ref, bs_ref, bn_ref, o_ref):
        deg = d0_ref[:, :1] + d1_ref[:, :1]
        inv = 1.0 / jnp.maximum(deg, 1.0)
        asum = a0_ref[...] + a1_ref[...]
        if sub_deg:
            asum = asum - d0_ref[...] - d1_ref[...]
        hn = asum * inv
        cat = jnp.concatenate([x_ref[...], hn], axis=1)
        wcat = jnp.concatenate([ws_ref[...], wn_ref[...]], axis=0)
        o_ref[...] = (
            jnp.dot(cat, wcat, preferred_element_type=jnp.float32)
            + bs_ref[...] + bn_ref[...]
        )

    return _tc_body


def _tc_combine(x, aggp, degp, w_self, w_neigh, b_self, b_neigh, sub_deg):
    return pl.pallas_call(
        _make_tc_body(sub_deg),
        grid=(NBLK,),
        in_specs=[
            pl.BlockSpec((BLK, D), lambda i: (i, 0)),
            pl.BlockSpec((BLK, D), lambda i: (i, 0)),
            pl.BlockSpec((BLK, D), lambda i: (i + NBLK, 0)),
            pl.BlockSpec((BLK, D), lambda i: (i, 0)),
            pl.BlockSpec((BLK, D), lambda i: (i + NBLK, 0)),
            pl.BlockSpec((D, D), lambda i: (0, 0)),
            pl.BlockSpec((D, D), lambda i: (0, 0)),
            pl.BlockSpec((1, D), lambda i: (0, 0)),
            pl.BlockSpec((1, D), lambda i: (0, 0)),
        ],
        out_specs=pl.BlockSpec((BLK, D), lambda i: (i, 0)),
        out_shape=jax.ShapeDtypeStruct((N, D), jnp.float32),
    )(x, aggp, aggp, degp, degp, w_self, w_neigh, b_self, b_neigh)


def kernel(x, edge_index, W_self1, b_self1, W_neigh1, b_neigh1,
           W_self2, b_self2, W_neigh2, b_neigh2):
    src = edge_index[0].reshape(NW, EPW)
    dst = edge_index[1].reshape(NW, NCHUNK, CHUNK)
    zeros = jnp.zeros((N, D), jnp.float32)

    agg1, degp = _sc_agg_deg(x, src, dst, zeros)
    h1 = _tc_combine(x, agg1, degp, W_self1, W_neigh1,
                     b_self1.reshape(1, D), b_neigh1.reshape(1, D),
                     sub_deg=True)
    agg2 = _sc_agg(h1, src, dst, zeros)
    h2 = _tc_combine(h1, agg2, degp, W_self2, W_neigh2,
                     b_self2.reshape(1, D), b_neigh2.reshape(1, D),
                     sub_deg=False)
    return h2


# speedup vs baseline: 1.0020x; 1.0020x over previous
import functools

import jax
import jax.numpy as jnp
from jax import lax
from jax.experimental import pallas as pl
from jax.experimental.pallas import tpu as pltpu
from jax.experimental.pallas import tpu_sc as plsc

N = 10000
E = 320000
D = 128

NC = 2
NS = 16
NW = NC * NS
EPW = E // NW
CHUNK = 80
NCHUNK = EPW // CHUNK
RPT = 624
TAIL = N - RPT * NS


def _striped_copy(s, copy):
    copy(s * RPT, RPT)

    @pl.when(s == NS - 1)
    def _():
        copy(NS * RPT, TAIL)

_MESH = plsc.VectorSubcoreMesh(core_axis_name="c", subcore_axis_name="s")


_SC_AGG_KW = dict(
    out_type=jax.ShapeDtypeStruct((NC * N, D), jnp.float32),
    mesh=_MESH,
    scratch_types=[
        pltpu.VMEM((EPW,), jnp.int32),
        pltpu.VMEM((NCHUNK, CHUNK), jnp.int32),
        pltpu.VMEM((CHUNK, D), jnp.float32),
        pltpu.VMEM((CHUNK, D), jnp.float32),
        pltpu.VMEM_SHARED((N, D), jnp.float32),
        pltpu.SemaphoreType.DMA,
        pltpu.SemaphoreType.DMA,
    ],
)


def _agg_step(h, srcv, dstv, aggsh, j, cur, csem, nxt, nsem):
    @pl.when(j + 1 < NCHUNK)
    def _():
        pltpu.async_copy(h.at[srcv.at[pl.ds((j + 1) * CHUNK, CHUNK)]], nxt, nsem)

    pltpu.make_async_copy(h.at[srcv.at[pl.ds(j * CHUNK, CHUNK)]], cur, csem).wait()
    pltpu.sync_copy(cur, aggsh.at[dstv.at[j]], add=True)


_DEG_WND = 64


def _agg_impl(h, src, dst, zeros, agg_out, deg_out,
              srcv, dstv, rows0, rows1, aggsh, gsem0, gsem1, ssem):
    c = lax.axis_index("c")
    s = lax.axis_index("s")
    wid = s * NC + c
    pltpu.sync_copy(src.at[wid], srcv)
    pltpu.sync_copy(dst.at[wid], dstv)
    _striped_copy(s, lambda r0, nr: pltpu.sync_copy(
        zeros.at[pl.ds(r0, nr)], aggsh.at[pl.ds(r0, nr)]))

    if deg_out is not None:
        def fill(i, carry):
            for k in range(D // 16):
                rows0[i, pl.ds(k * 16, 16)] = jnp.ones((16,), jnp.float32)
            return carry

        lax.fori_loop(0, CHUNK, fill, 0)
        plsc.subcore_barrier()

        def dstep(j, carry):
            pltpu.async_copy(rows0, aggsh.at[dstv.at[j]], ssem, add=True)

            @pl.when(j >= _DEG_WND)
            def _():
                pltpu.make_async_copy(rows0, aggsh.at[dstv.at[0]], ssem).wait()

            return carry

        lax.fori_loop(0, NCHUNK, dstep, 0)

        def ddrain(j, carry):
            pltpu.make_async_copy(rows0, aggsh.at[dstv.at[0]], ssem).wait()
            return carry

        lax.fori_loop(0, _DEG_WND, ddrain, 0)
        plsc.subcore_barrier()
        _striped_copy(s, lambda r0, nr: pltpu.sync_copy(
            aggsh.at[pl.ds(r0, nr)], deg_out.at[pl.ds(c * N + r0, nr)]))

    pltpu.async_copy(h.at[srcv.at[pl.ds(0, CHUNK)]], rows0, gsem0)
    plsc.subcore_barrier()

    def pair(p, carry):
        j = 2 * p
        _agg_step(h, srcv, dstv, aggsh, j, rows0, gsem0, rows1, gsem1)
        _agg_step(h, srcv, dstv, aggsh, j + 1, rows1, gsem1, rows0, gsem0)
        return carry

    lax.fori_loop(0, NCHUNK // 2, pair, 0)
    if NCHUNK % 2:
        _agg_step(h, srcv, dstv, aggsh, NCHUNK - 1, rows0, gsem0, rows1, gsem1)
    plsc.subcore_barrier()
    _striped_copy(s, lambda r0, nr: pltpu.sync_copy(
        aggsh.at[pl.ds(r0, nr)], agg_out.at[pl.ds(c * N + r0, nr)]))


def _sc_agg_body(h, src, dst, zeros, agg_out,
                 srcv, dstv, rows0, rows1, aggsh, gsem0, gsem1):
    _agg_impl(h, src, dst, zeros, agg_out, None,
              srcv, dstv, rows0, rows1, aggsh, gsem0, gsem1, None)


def _sc_agg_deg_body(h, src, dst, zeros, agg_out, deg_out,
                     srcv, dstv, rows0, rows1, aggsh, gsem0, gsem1, ssem):
    _agg_impl(h, src, dst, zeros, agg_out, deg_out,
              srcv, dstv, rows0, rows1, aggsh, gsem0, gsem1, ssem)


_sc_agg = pl.kernel(_sc_agg_body, **_SC_AGG_KW)

_SC_AGG_DEG_KW = dict(
    out_type=(
        jax.ShapeDtypeStruct((NC * N, D), jnp.float32),
        jax.ShapeDtypeStruct((NC * N, D), jnp.float32),
    ),
    mesh=_MESH,
    scratch_types=list(_SC_AGG_KW["scratch_types"]) + [pltpu.SemaphoreType.DMA],
)

_sc_agg_deg = pl.kernel(_sc_agg_deg_body, **_SC_AGG_DEG_KW)


BLK = 2000
NBLK = N // BLK


def _make_tc_body(sub_deg):
    def _tc_body(x_ref, a0_ref, a1_ref, d0_ref, d1_ref,
                 ws_ref, wn_ref, bs_ref, bn_ref, o_ref):
        deg = d0_ref[:, :1] + d1_ref[:, :1]
        inv = 1.0 / jnp.maximum(deg, 1.0)
        asum = a0_ref[...] + a1_ref[...]
        if sub_deg:
            asum = asum - d0_ref[...] - d1_ref[...]
        hn = asum * inv
        cat = jnp.concatenate([x_ref[...], hn], axis=1)
        wcat = jnp.concatenate([ws_ref[...], wn_ref[...]], axis=0)
        o_ref[...] = (
            jnp.dot(cat, wcat, preferred_element_type=jnp.float32)
            + bs_ref[...] + bn_ref[...]
        )

    return _tc_body


def _tc_combine(x, aggp, degp, w_self, w_neigh, b_self, b_neigh, sub_deg):
    return pl.pallas_call(
        _make_tc_body(sub_deg),
        grid=(NBLK,),
        in_specs=[
            pl.BlockSpec((BLK, D), lambda i: (i, 0)),
            pl.BlockSpec((BLK, D), lambda i: (i, 0)),
            pl.BlockSpec((BLK, D), lambda i: (i + NBLK, 0)),
            pl.BlockSpec((BLK, D), lambda i: (i, 0)),
            pl.BlockSpec((BLK, D), lambda i: (i + NBLK, 0)),
            pl.BlockSpec((D, D), lambda i: (0, 0)),
            pl.BlockSpec((D, D), lambda i: (0, 0)),
            pl.BlockSpec((1, D), lambda i: (0, 0)),
            pl.BlockSpec((1, D), lambda i: (0, 0)),
        ],
        out_specs=pl.BlockSpec((BLK, D), lambda i: (i, 0)),
        out_shape=jax.ShapeDtypeStruct((N, D), jnp.float32),
    )(x, aggp, aggp, degp, degp, w_self, w_neigh, b_self, b_neigh)


def kernel(x, edge_index, W_self1, b_self1, W_neigh1, b_neigh1,
           W_self2, b_self2, W_neigh2, b_neigh2):
    src = edge_index[0].reshape(NW, EPW)
    dst = edge_index[1].reshape(NW, NCHUNK, CHUNK)
    zeros = jnp.zeros((N, D), jnp.float32)

    agg1, degp = _sc_agg_deg(x, src, dst, zeros)
    h1 = _tc_combine(x, agg1, degp, W_self1, W_neigh1,
                     b_self1.reshape(1, D), b_neigh1.reshape(1, D),
                     sub_deg=True)
    agg2 = _sc_agg(h1, src, dst, zeros)
    h2 = _tc_combine(h1, agg2, degp, W_self2, W_neigh2,
                     b_self2.reshape(1, D), b_neigh2.reshape(1, D),
                     sub_deg=False)
    return h2


# speedup vs baseline: 1.0104x; 1.0084x over previous
import functools

import jax
import jax.numpy as jnp
from jax import lax
from jax.experimental import pallas as pl
from jax.experimental.pallas import tpu as pltpu
from jax.experimental.pallas import tpu_sc as plsc

N = 10000
E = 320000
D = 128

NC = 2
NS = 16
NW = NC * NS
EPW = E // NW
CHUNK = 80
NCHUNK = EPW // CHUNK
RPT = 624
TAIL = N - RPT * NS


def _striped_copy(s, copy):
    copy(s * RPT, RPT)

    @pl.when(s == NS - 1)
    def _():
        copy(NS * RPT, TAIL)

_MESH = plsc.VectorSubcoreMesh(core_axis_name="c", subcore_axis_name="s")


_SC_AGG_KW = dict(
    out_type=jax.ShapeDtypeStruct((NC * N, D), jnp.float32),
    mesh=_MESH,
    scratch_types=[
        pltpu.VMEM((EPW,), jnp.int32),
        pltpu.VMEM((NCHUNK, CHUNK), jnp.int32),
        pltpu.VMEM((CHUNK, D), jnp.float32),
        pltpu.VMEM((CHUNK, D), jnp.float32),
        pltpu.VMEM_SHARED((N, D), jnp.float32),
        pltpu.SemaphoreType.DMA,
        pltpu.SemaphoreType.DMA,
    ],
)


def _agg_step(h, srcv, dstv, aggsh, j, cur, csem, nxt, nsem):
    @pl.when(j + 1 < NCHUNK)
    def _():
        pltpu.async_copy(h.at[srcv.at[pl.ds((j + 1) * CHUNK, CHUNK)]], nxt, nsem)

    pltpu.make_async_copy(h.at[srcv.at[pl.ds(j * CHUNK, CHUNK)]], cur, csem).wait()
    pltpu.sync_copy(cur, aggsh.at[dstv.at[j]], add=True)


_DEG_WND = 64


def _agg_impl(h, src, dst, zeros, agg_out, deg_out,
              srcv, dstv, rows0, rows1, aggsh, gsem0, gsem1, ssem):
    c = lax.axis_index("c")
    s = lax.axis_index("s")
    wid = s * NC + c
    pltpu.async_copy(src.at[wid], srcv, gsem0)
    pltpu.async_copy(dst.at[wid], dstv, gsem1)
    _striped_copy(s, lambda r0, nr: pltpu.sync_copy(
        zeros.at[pl.ds(r0, nr)], aggsh.at[pl.ds(r0, nr)]))

    if deg_out is not None:
        def fill(i, carry):
            for k in range(D // 16):
                rows0[i, pl.ds(k * 16, 16)] = jnp.ones((16,), jnp.float32)
            return carry

        lax.fori_loop(0, CHUNK, fill, 0)

    pltpu.make_async_copy(src.at[wid], srcv, gsem0).wait()
    pltpu.make_async_copy(dst.at[wid], dstv, gsem1).wait()

    if deg_out is not None:
        plsc.subcore_barrier()

        def dstep(j, carry):
            pltpu.async_copy(rows0, aggsh.at[dstv.at[j]], ssem, add=True)

            @pl.when(j >= _DEG_WND)
            def _():
                pltpu.make_async_copy(rows0, aggsh.at[dstv.at[0]], ssem).wait()

            return carry

        lax.fori_loop(0, NCHUNK, dstep, 0)

        def ddrain(j, carry):
            pltpu.make_async_copy(rows0, aggsh.at[dstv.at[0]], ssem).wait()
            return carry

        lax.fori_loop(0, _DEG_WND, ddrain, 0)
        plsc.subcore_barrier()
        _striped_copy(s, lambda r0, nr: pltpu.sync_copy(
            aggsh.at[pl.ds(r0, nr)], deg_out.at[pl.ds(c * N + r0, nr)]))

    pltpu.async_copy(h.at[srcv.at[pl.ds(0, CHUNK)]], rows0, gsem0)
    plsc.subcore_barrier()

    def pair(p, carry):
        j = 2 * p
        _agg_step(h, srcv, dstv, aggsh, j, rows0, gsem0, rows1, gsem1)
        _agg_step(h, srcv, dstv, aggsh, j + 1, rows1, gsem1, rows0, gsem0)
        return carry

    lax.fori_loop(0, NCHUNK // 2, pair, 0)
    if NCHUNK % 2:
        _agg_step(h, srcv, dstv, aggsh, NCHUNK - 1, rows0, gsem0, rows1, gsem1)
    plsc.subcore_barrier()
    _striped_copy(s, lambda r0, nr: pltpu.sync_copy(
        aggsh.at[pl.ds(r0, nr)], agg_out.at[pl.ds(c * N + r0, nr)]))


def _sc_agg_body(h, src, dst, zeros, agg_out,
                 srcv, dstv, rows0, rows1, aggsh, gsem0, gsem1):
    _agg_impl(h, src, dst, zeros, agg_out, None,
              srcv, dstv, rows0, rows1, aggsh, gsem0, gsem1, None)


def _sc_agg_deg_body(h, src, dst, zeros, agg_out, deg_out,
                     srcv, dstv, rows0, rows1, aggsh, gsem0, gsem1, ssem):
    _agg_impl(h, src, dst, zeros, agg_out, deg_out,
              srcv, dstv, rows0, rows1, aggsh, gsem0, gsem1, ssem)


_sc_agg = pl.kernel(_sc_agg_body, **_SC_AGG_KW)

_SC_AGG_DEG_KW = dict(
    out_type=(
        jax.ShapeDtypeStruct((NC * N, D), jnp.float32),
        jax.ShapeDtypeStruct((NC * N, D), jnp.float32),
    ),
    mesh=_MESH,
    scratch_types=list(_SC_AGG_KW["scratch_types"]) + [pltpu.SemaphoreType.DMA],
)

_sc_agg_deg = pl.kernel(_sc_agg_deg_body, **_SC_AGG_DEG_KW)


BLK = 2000
NBLK = N // BLK


def _make_tc_body(sub_deg):
    def _tc_body(x_ref, a0_ref, a1_ref, d0_ref, d1_ref,
                 ws_ref, wn_ref, bs_ref, bn_ref, o_ref):
        deg = d0_ref[:, :1] + d1_ref[:, :1]
        inv = 1.0 / jnp.maximum(deg, 1.0)
        asum = a0_ref[...] + a1_ref[...]
        if sub_deg:
            asum = asum - d0_ref[...] - d1_ref[...]
        hn = asum * inv
        cat = jnp.concatenate([x_ref[...], hn], axis=1)
        wcat = jnp.concatenate([ws_ref[...], wn_ref[...]], axis=0)
        o_ref[...] = (
            jnp.dot(cat, wcat, preferred_element_type=jnp.float32)
            + bs_ref[...] + bn_ref[...]
        )

    return _tc_body


def _tc_combine(x, aggp, degp, w_self, w_neigh, b_self, b_neigh, sub_deg):
    return pl.pallas_call(
        _make_tc_body(sub_deg),
        grid=(NBLK,),
        in_specs=[
            pl.BlockSpec((BLK, D), lambda i: (i, 0)),
            pl.BlockSpec((BLK, D), lambda i: (i, 0)),
            pl.BlockSpec((BLK, D), lambda i: (i + NBLK, 0)),
            pl.BlockSpec((BLK, D), lambda i: (i, 0)),
            pl.BlockSpec((BLK, D), lambda i: (i + NBLK, 0)),
            pl.BlockSpec((D, D), lambda i: (0, 0)),
            pl.BlockSpec((D, D), lambda i: (0, 0)),
            pl.BlockSpec((1, D), lambda i: (0, 0)),
            pl.BlockSpec((1, D), lambda i: (0, 0)),
        ],
        out_specs=pl.BlockSpec((BLK, D), lambda i: (i, 0)),
        out_shape=jax.ShapeDtypeStruct((N, D), jnp.float32),
    )(x, aggp, aggp, degp, degp, w_self, w_neigh, b_self, b_neigh)


def kernel(x, edge_index, W_self1, b_self1, W_neigh1, b_neigh1,
           W_self2, b_self2, W_neigh2, b_neigh2):
    src = edge_index[0].reshape(NW, EPW)
    dst = edge_index[1].reshape(NW, NCHUNK, CHUNK)
    zeros = jnp.zeros((N, D), jnp.float32)

    agg1, degp = _sc_agg_deg(x, src, dst, zeros)
    h1 = _tc_combine(x, agg1, degp, W_self1, W_neigh1,
                     b_self1.reshape(1, D), b_neigh1.reshape(1, D),
                     sub_deg=True)
    agg2 = _sc_agg(h1, src, dst, zeros)
    h2 = _tc_combine(h1, agg2, degp, W_self2, W_neigh2,
                     b_self2.reshape(1, D), b_neigh2.reshape(1, D),
                     sub_deg=False)
    return h2


# speedup vs baseline: 1.0110x; 1.0006x over previous
import functools

import jax
import jax.numpy as jnp
from jax import lax
from jax.experimental import pallas as pl
from jax.experimental.pallas import tpu as pltpu
from jax.experimental.pallas import tpu_sc as plsc

N = 10000
E = 320000
D = 128

NC = 2
NS = 16
NW = NC * NS
EPW = E // NW
CHUNK = 80
NCHUNK = EPW // CHUNK
RPT = 624
TAIL = N - RPT * NS


def _striped_copy(s, copy):
    copy(s * RPT, RPT)

    @pl.when(s == NS - 1)
    def _():
        copy(NS * RPT, TAIL)

_MESH = plsc.VectorSubcoreMesh(core_axis_name="c", subcore_axis_name="s")


_SC_AGG_KW = dict(
    out_type=jax.ShapeDtypeStruct((NC * N, D), jnp.float32),
    mesh=_MESH,
    scratch_types=[
        pltpu.VMEM((EPW,), jnp.int32),
        pltpu.VMEM((NCHUNK, CHUNK), jnp.int32),
        pltpu.VMEM((CHUNK, D), jnp.float32),
        pltpu.VMEM((CHUNK, D), jnp.float32),
        pltpu.VMEM_SHARED((N, D), jnp.float32),
        pltpu.SemaphoreType.DMA,
        pltpu.SemaphoreType.DMA,
    ],
)


def _agg_step(h, srcv, dstv, aggsh, j, cur, csem, nxt, nsem):
    @pl.when(j + 1 < NCHUNK)
    def _():
        pltpu.async_copy(h.at[srcv.at[pl.ds((j + 1) * CHUNK, CHUNK)]], nxt, nsem)

    pltpu.make_async_copy(h.at[srcv.at[pl.ds(j * CHUNK, CHUNK)]], cur, csem).wait()
    pltpu.sync_copy(cur, aggsh.at[dstv.at[j]], add=True)


_DEG_WND = 64


def _agg_impl(h, src, dst, zeros, agg_out, deg_out,
              srcv, dstv, rows0, rows1, aggsh, gsem0, gsem1, ssem):
    c = lax.axis_index("c")
    s = lax.axis_index("s")
    wid = s * NC + c
    pltpu.async_copy(src.at[wid], srcv, gsem0)
    pltpu.async_copy(dst.at[wid], dstv, gsem1)

    if deg_out is None:
        pltpu.make_async_copy(src.at[wid], srcv, gsem0).wait()
        pltpu.async_copy(h.at[srcv.at[pl.ds(0, CHUNK)]], rows0, gsem0)

    _striped_copy(s, lambda r0, nr: pltpu.sync_copy(
        zeros.at[pl.ds(r0, nr)], aggsh.at[pl.ds(r0, nr)]))

    if deg_out is not None:
        def fill(i, carry):
            for k in range(D // 16):
                rows0[i, pl.ds(k * 16, 16)] = jnp.ones((16,), jnp.float32)
            return carry

        lax.fori_loop(0, CHUNK, fill, 0)
        pltpu.make_async_copy(src.at[wid], srcv, gsem0).wait()

    pltpu.make_async_copy(dst.at[wid], dstv, gsem1).wait()

    if deg_out is not None:
        plsc.subcore_barrier()

        def dstep(j, carry):
            pltpu.async_copy(rows0, aggsh.at[dstv.at[j]], ssem, add=True)

            @pl.when(j >= _DEG_WND)
            def _():
                pltpu.make_async_copy(rows0, aggsh.at[dstv.at[0]], ssem).wait()

            return carry

        lax.fori_loop(0, NCHUNK, dstep, 0)

        def ddrain(j, carry):
            pltpu.make_async_copy(rows0, aggsh.at[dstv.at[0]], ssem).wait()
            return carry

        lax.fori_loop(0, _DEG_WND, ddrain, 0)
        pltpu.async_copy(h.at[srcv.at[pl.ds(0, CHUNK)]], rows0, gsem0)
        plsc.subcore_barrier()
        _striped_copy(s, lambda r0, nr: pltpu.sync_copy(
            aggsh.at[pl.ds(r0, nr)], deg_out.at[pl.ds(c * N + r0, nr)]))

    plsc.subcore_barrier()

    def pair(p, carry):
        j = 2 * p
        _agg_step(h, srcv, dstv, aggsh, j, rows0, gsem0, rows1, gsem1)
        _agg_step(h, srcv, dstv, aggsh, j + 1, rows1, gsem1, rows0, gsem0)
        return carry

    lax.fori_loop(0, NCHUNK // 2, pair, 0)
    if NCHUNK % 2:
        _agg_step(h, srcv, dstv, aggsh, NCHUNK - 1, rows0, gsem0, rows1, gsem1)
    plsc.subcore_barrier()
    _striped_copy(s, lambda r0, nr: pltpu.sync_copy(
        aggsh.at[pl.ds(r0, nr)], agg_out.at[pl.ds(c * N + r0, nr)]))


def _sc_agg_body(h, src, dst, zeros, agg_out,
                 srcv, dstv, rows0, rows1, aggsh, gsem0, gsem1):
    _agg_impl(h, src, dst, zeros, agg_out, None,
              srcv, dstv, rows0, rows1, aggsh, gsem0, gsem1, None)


def _sc_agg_deg_body(h, src, dst, zeros, agg_out, deg_out,
                     srcv, dstv, rows0, rows1, aggsh, gsem0, gsem1, ssem):
    _agg_impl(h, src, dst, zeros, agg_out, deg_out,
              srcv, dstv, rows0, rows1, aggsh, gsem0, gsem1, ssem)


_sc_agg = pl.kernel(_sc_agg_body, **_SC_AGG_KW)

_SC_AGG_DEG_KW = dict(
    out_type=(
        jax.ShapeDtypeStruct((NC * N, D), jnp.float32),
        jax.ShapeDtypeStruct((NC * N, D), jnp.float32),
    ),
    mesh=_MESH,
    scratch_types=list(_SC_AGG_KW["scratch_types"]) + [pltpu.SemaphoreType.DMA],
)

_sc_agg_deg = pl.kernel(_sc_agg_deg_body, **_SC_AGG_DEG_KW)


BLK = 2000
NBLK = N // BLK


def _make_tc_body(sub_deg):
    def _tc_body(x_ref, a0_ref, a1_ref, d0_ref, d1_ref,
                 ws_ref, wn_ref, bs_ref, bn_ref, o_ref):
        deg = d0_ref[:, :1] + d1_ref[:, :1]
        inv = 1.0 / jnp.maximum(deg, 1.0)
        asum = a0_ref[...] + a1_ref[...]
        if sub_deg:
            asum = asum - d0_ref[...] - d1_ref[...]
        hn = asum * inv
        cat = jnp.concatenate([x_ref[...], hn], axis=1)
        wcat = jnp.concatenate([ws_ref[...], wn_ref[...]], axis=0)
        o_ref[...] = (
            jnp.dot(cat, wcat, preferred_element_type=jnp.float32)
            + bs_ref[...] + bn_ref[...]
        )

    return _tc_body


def _tc_combine(x, aggp, degp, w_self, w_neigh, b_self, b_neigh, sub_deg):
    return pl.pallas_call(
        _make_tc_body(sub_deg),
        grid=(NBLK,),
        in_specs=[
            pl.BlockSpec((BLK, D), lambda i: (i, 0)),
            pl.BlockSpec((BLK, D), lambda i: (i, 0)),
            pl.BlockSpec((BLK, D), lambda i: (i + NBLK, 0)),
            pl.BlockSpec((BLK, D), lambda i: (i, 0)),
            pl.BlockSpec((BLK, D), lambda i: (i + NBLK, 0)),
            pl.BlockSpec((D, D), lambda i: (0, 0)),
            pl.BlockSpec((D, D), lambda i: (0, 0)),
            pl.BlockSpec((1, D), lambda i: (0, 0)),
            pl.BlockSpec((1, D), lambda i: (0, 0)),
        ],
        out_specs=pl.BlockSpec((BLK, D), lambda i: (i, 0)),
        out_shape=jax.ShapeDtypeStruct((N, D), jnp.float32),
    )(x, aggp, aggp, degp, degp, w_self, w_neigh, b_self, b_neigh)


def kernel(x, edge_index, W_self1, b_self1, W_neigh1, b_neigh1,
           W_self2, b_self2, W_neigh2, b_neigh2):
    src = edge_index[0].reshape(NW, EPW)
    dst = edge_index[1].reshape(NW, NCHUNK, CHUNK)
    zeros = jnp.zeros((N, D), jnp.float32)

    agg1, degp = _sc_agg_deg(x, src, dst, zeros)
    h1 = _tc_combine(x, agg1, degp, W_self1, W_neigh1,
                     b_self1.reshape(1, D), b_neigh1.reshape(1, D),
                     sub_deg=True)
    agg2 = _sc_agg(h1, src, dst, zeros)
    h2 = _tc_combine(h1, agg2, degp, W_self2, W_neigh2,
                     b_self2.reshape(1, D), b_neigh2.reshape(1, D),
                     sub_deg=False)
    return h2
